# baseline (speedup 1.0000x reference)
import functools

import jax
import jax.numpy as jnp
from jax import lax
from jax.experimental import pallas as pl
from jax.experimental.pallas import tpu as pltpu
from jax.experimental.pallas import tpu_sc as plsc


@functools.cache
def _build(B, V, D):
    info = plsc.get_sparse_core_info()
    NC, NS = info.num_cores, info.num_subcores
    NW = NC * NS
    b_per_w = B // NW
    stage = max(64, -(-V // NS))
    mesh = plsc.VectorSubcoreMesh(core_axis_name="c", subcore_axis_name="s")

    sizes = [32, 32, 64, 64, 128, 128, 64]
    assert sum(sizes) == b_per_w
    n_hbm = 2
    offs = [sum(sizes[:i]) for i in range(len(sizes))]
    n_chunks = len(sizes)

    @functools.partial(
        pl.kernel,
        mesh=mesh,
        out_type=jax.ShapeDtypeStruct((B, D), jnp.float32),
        scratch_types=[
            pltpu.VMEM((b_per_w,), jnp.int32),
            pltpu.VMEM((b_per_w, D), jnp.float32),
            pltpu.VMEM_SHARED((V, D), jnp.float32),
            pltpu.SemaphoreType.DMA,
            pltpu.SemaphoreType.DMA((n_chunks,)),
            pltpu.SemaphoreType.DMA,
        ],
    )
    def k(idx_hbm, table_hbm, out_hbm, idx_v, rows_v, table_sh, isem, gsem, wsem):
        cid = lax.axis_index("c")
        sid = lax.axis_index("s")
        wid = sid * NC + cid
        base = wid * b_per_w
        pltpu.async_copy(idx_hbm.at[wid], idx_v, isem).wait()

        def gather(j, src):
            return pltpu.async_copy(
                src.at[idx_v.at[pl.ds(offs[j], sizes[j])]],
                rows_v.at[pl.ds(offs[j], sizes[j])],
                gsem.at[j],
            )

        gathers = [gather(j, table_hbm) for j in range(n_hbm)]
        row0 = jnp.minimum(sid * stage, V - stage)
        pltpu.sync_copy(
            table_hbm.at[pl.ds(row0, stage)], table_sh.at[pl.ds(row0, stage)]
        )
        plsc.subcore_barrier()
        gathers += [gather(j, table_sh) for j in range(n_hbm, n_chunks)]
        writes = []
        for j in range(n_chunks):
            gathers[j].wait()
            writes.append(
                pltpu.async_copy(
                    rows_v.at[pl.ds(offs[j], sizes[j])],
                    out_hbm.at[pl.ds(base + offs[j], sizes[j])],
                    wsem,
                )
            )
        for w in writes:
            w.wait()

    def run(x, table):
        idx = x.astype(jnp.int32).reshape(NW, b_per_w)
        out = k(idx, table)
        return out.reshape(B, 1, D)

    return run


def kernel(x, table):
    B = x.shape[0]
    V, D = table.shape
    return _build(B, V, D)(x, table)


# speedup vs baseline: 1.0104x; 1.0104x over previous
import functools

import jax
import jax.numpy as jnp
from jax import lax
from jax.experimental import pallas as pl
from jax.experimental.pallas import tpu as pltpu
from jax.experimental.pallas import tpu_sc as plsc


@functools.cache
def _build(B, V, D):
    info = plsc.get_sparse_core_info()
    NC, NS = info.num_cores, info.num_subcores
    NW = NC * NS
    b_per_w = B // NW
    stage = max(64, -(-V // NS))
    mesh = plsc.VectorSubcoreMesh(core_axis_name="c", subcore_axis_name="s")

    chunk = 64
    n_chunks = b_per_w // chunk
    n_hbm = 2

    @functools.partial(
        pl.kernel,
        mesh=mesh,
        out_type=jax.ShapeDtypeStruct((B, D), jnp.float32),
        scratch_types=[
            pltpu.VMEM((n_chunks, chunk), jnp.int32),
            pltpu.VMEM((n_chunks, chunk, D), jnp.float32),
            pltpu.VMEM_SHARED((V, D), jnp.float32),
            pltpu.SemaphoreType.DMA,
            pltpu.SemaphoreType.DMA((n_chunks,)),
            pltpu.SemaphoreType.DMA,
        ],
    )
    def k(idx_hbm, table_hbm, out_hbm, idx_v, rows_v, table_sh, isem, gsem, wsem):
        cid = lax.axis_index("c")
        sid = lax.axis_index("s")
        wid = sid * NC + cid
        base = wid * b_per_w
        pltpu.async_copy(idx_hbm.at[wid], idx_v, isem).wait()

        def gather(j, src):
            return pltpu.async_copy(
                src.at[idx_v.at[j]], rows_v.at[j], gsem.at[j]
            )

        gathers = [gather(j, table_hbm) for j in range(n_hbm)]
        row0 = jnp.minimum(sid * stage, V - stage)
        pltpu.sync_copy(
            table_hbm.at[pl.ds(row0, stage)], table_sh.at[pl.ds(row0, stage)]
        )
        plsc.subcore_barrier()
        gathers += [gather(j, table_sh) for j in range(n_hbm, n_chunks)]
        writes = []
        for j in range(n_chunks):
            gathers[j].wait()
            writes.append(
                pltpu.async_copy(
                    rows_v.at[j], out_hbm.at[pl.ds(base + j * chunk, chunk)], wsem
                )
            )
        for w in writes:
            w.wait()

    def run(x, table):
        idx = x.astype(jnp.int32).reshape(NW, n_chunks, chunk)
        out = k(idx, table)
        return out.reshape(B, 1, D)

    return run


def kernel(x, table):
    B = x.shape[0]
    V, D = table.shape
    return _build(B, V, D)(x, table)


# speedup vs baseline: 1.0381x; 1.0274x over previous
import functools

import jax
import jax.numpy as jnp
from jax import lax
from jax.experimental import pallas as pl
from jax.experimental.pallas import tpu as pltpu
from jax.experimental.pallas import tpu_sc as plsc


@functools.cache
def _build(B, V, D):
    info = plsc.get_sparse_core_info()
    NC, NS = info.num_cores, info.num_subcores
    NW = NC * NS
    b_per_w = B // NW
    stage = max(64, -(-V // NS))
    mesh = plsc.VectorSubcoreMesh(core_axis_name="c", subcore_axis_name="s")

    chunk = 64
    n_chunks = b_per_w // chunk

    @functools.partial(
        pl.kernel,
        mesh=mesh,
        out_type=jax.ShapeDtypeStruct((B, D), jnp.float32),
        scratch_types=[
            pltpu.VMEM((n_chunks, chunk), jnp.int32),
            pltpu.VMEM((n_chunks, chunk, D), jnp.float32),
            pltpu.VMEM_SHARED((V, D), jnp.float32),
            pltpu.SemaphoreType.DMA,
            pltpu.SemaphoreType.DMA((n_chunks,)),
            pltpu.SemaphoreType.DMA,
        ],
    )
    def k(idx_hbm, table_hbm, out_hbm, idx_v, rows_v, table_sh, isem, gsem, wsem):
        cid = lax.axis_index("c")
        sid = lax.axis_index("s")
        wid = sid * NC + cid
        base = wid * b_per_w
        row0 = jnp.minimum(sid * stage, V - stage)
        icopy = pltpu.async_copy(idx_hbm.at[wid], idx_v, isem)
        pltpu.sync_copy(
            table_hbm.at[pl.ds(row0, stage)], table_sh.at[pl.ds(row0, stage)]
        )
        plsc.subcore_barrier()
        icopy.wait()
        gathers = [
            pltpu.async_copy(table_sh.at[idx_v.at[j]], rows_v.at[j], gsem.at[j])
            for j in range(n_chunks)
        ]
        writes = []
        for j in range(n_chunks):
            gathers[j].wait()
            writes.append(
                pltpu.async_copy(
                    rows_v.at[j], out_hbm.at[pl.ds(base + j * chunk, chunk)], wsem
                )
            )
        for w in writes:
            w.wait()

    def run(x, table):
        idx = x.astype(jnp.int32).reshape(NW, n_chunks, chunk)
        out = k(idx, table)
        return out.reshape(B, 1, D)

    return run


def kernel(x, table):
    B = x.shape[0]
    V, D = table.shape
    return _build(B, V, D)(x, table)


# speedup vs baseline: 1.0390x; 1.0009x over previous
import functools

import jax
import jax.numpy as jnp
from jax import lax
from jax.experimental import pallas as pl
from jax.experimental.pallas import tpu as pltpu
from jax.experimental.pallas import tpu_sc as plsc


@functools.cache
def _build(B, V, D):
    info = plsc.get_sparse_core_info()
    NC, NS = info.num_cores, info.num_subcores
    NW = NC * NS
    b_per_w = B // NW
    stage = max(64, -(-V // NS))
    mesh = plsc.VectorSubcoreMesh(core_axis_name="c", subcore_axis_name="s")

    chunk = 64
    n_chunks = b_per_w // chunk

    @functools.partial(
        pl.kernel,
        mesh=mesh,
        out_type=jax.ShapeDtypeStruct((B, D), jnp.float32),
        scratch_types=[
            pltpu.VMEM((n_chunks, chunk), jnp.int32),
            pltpu.VMEM((n_chunks // 2, 2 * chunk, D), jnp.float32),
            pltpu.VMEM_SHARED((V, D), jnp.float32),
            pltpu.SemaphoreType.DMA,
            pltpu.SemaphoreType.DMA((n_chunks,)),
            pltpu.SemaphoreType.DMA,
        ],
    )
    def k(idx_hbm, table_hbm, out_hbm, idx_v, rows_v, table_sh, isem, gsem, wsem):
        cid = lax.axis_index("c")
        sid = lax.axis_index("s")
        wid = sid * NC + cid
        base = wid * b_per_w
        row0 = jnp.minimum(sid * stage, V - stage)
        icopy = pltpu.async_copy(idx_hbm.at[wid], idx_v, isem)
        pltpu.sync_copy(
            table_hbm.at[pl.ds(row0, stage)], table_sh.at[pl.ds(row0, stage)]
        )
        plsc.subcore_barrier()
        icopy.wait()
        gathers = [
            pltpu.async_copy(
                table_sh.at[idx_v.at[j]],
                rows_v.at[j // 2, pl.ds((j % 2) * chunk, chunk)],
                gsem.at[j],
            )
            for j in range(n_chunks)
        ]
        writes = []
        for i in range(n_chunks // 2):
            gathers[2 * i].wait()
            gathers[2 * i + 1].wait()
            writes.append(
                pltpu.async_copy(
                    rows_v.at[i],
                    out_hbm.at[pl.ds(base + i * 2 * chunk, 2 * chunk)],
                    wsem,
                )
            )
        for w in writes:
            w.wait()

    def run(x, table):
        idx = x.astype(jnp.int32).reshape(NW, n_chunks, chunk)
        out = k(idx, table)
        return out.reshape(B, 1, D)

    return run


def kernel(x, table):
    B = x.shape[0]
    V, D = table.shape
    return _build(B, V, D)(x, table)
